# baseline (speedup 1.0000x reference)
import functools

import jax
import jax.numpy as jnp
from jax import lax
from jax.experimental import pallas as pl
from jax.experimental.pallas import tpu as pltpu
from jax.experimental.pallas import tpu_sc as plsc

B, L, D = 1024, 50, 128
LP = 56
NP = B * LP
VOCAB = 100000

NC, NS = 2, 16
NW = NC * NS
TOK_PER_W = NP // NW
CHUNK = 64
NCHUNK = TOK_PER_W // CHUNK

EPS = 1e-5
C1, C2, C3, C4 = 2.0, 4.0 / 3.0, 8.0 / 7.0, 16.0 / 15.0


def _gather_body(x_hbm, table_hbm, out_hbm, idx_v, rows_v, sem0, sem1):
    wid = lax.axis_index("s") * NC + lax.axis_index("c")
    base = wid * TOK_PER_W

    def start(j, slot):
        off = base + j * CHUNK
        pltpu.sync_copy(x_hbm.at[pl.ds(off, CHUNK)], idx_v.at[slot])
        sem = sem0 if slot == 0 else sem1
        return pltpu.async_copy(table_hbm.at[idx_v.at[slot]], rows_v.at[slot], sem)

    dma = start(0, 0)
    for j in range(NCHUNK):
        slot = j % 2
        if j + 1 < NCHUNK:
            nxt = start(j + 1, (j + 1) % 2)
        dma.wait()
        off = base + j * CHUNK
        pltpu.sync_copy(rows_v.at[slot], out_hbm.at[pl.ds(off, CHUNK)])
        if j + 1 < NCHUNK:
            dma = nxt


def _sc_gather(x_flat, table):
    mesh = plsc.VectorSubcoreMesh(core_axis_name="c", subcore_axis_name="s")
    fn = pl.kernel(
        _gather_body,
        mesh=mesh,
        out_type=jax.ShapeDtypeStruct((NP, D), jnp.float32),
        scratch_types=[
            pltpu.VMEM((2, CHUNK), jnp.int32),
            pltpu.VMEM((2, CHUNK, D), jnp.float32),
            pltpu.SemaphoreType.DMA,
            pltpu.SemaphoreType.DMA,
        ],
    )
    return fn(x_flat, table)


BB = 64


def _lif_body(rows_ref, pos_ref, gam_ref, bet_ref, out_ref, y_ref):
    t = pl.program_id(1)

    @pl.when(t == 0)
    def _():
        h = rows_ref[...] + pos_ref[...]
        mu = jnp.mean(h, axis=-1, keepdims=True)
        var = jnp.mean((h - mu) ** 2, axis=-1, keepdims=True)
        y_ref[...] = (h - mu) * lax.rsqrt(var + EPS) * gam_ref[...] + bet_ref[...]

    y = y_ref[...]
    a = y >= C1
    b = y >= C2
    c = y >= C3
    d = y >= C4
    one = jnp.float32(1.0)
    zero = jnp.float32(0.0)
    s0 = jnp.where(a, one, zero)
    s1 = jnp.where(b, one, zero)
    s2 = jnp.where(a | (c & ~b), one, zero)
    s3 = jnp.where(b | (d & ~c), one, zero)
    s = jnp.where(t == 0, s0, jnp.where(t == 1, s1, jnp.where(t == 2, s2, s3)))
    out_ref[0] = s[:, :L, :]


def _tc_lif(rows3, pos, gamma, beta):
    grid = (B // BB, 4)
    return pl.pallas_call(
        _lif_body,
        grid=grid,
        in_specs=[
            pl.BlockSpec((BB, LP, D), lambda i, t: (i, 0, 0)),
            pl.BlockSpec((1, LP, D), lambda i, t: (0, 0, 0)),
            pl.BlockSpec((1, 1, D), lambda i, t: (0, 0, 0)),
            pl.BlockSpec((1, 1, D), lambda i, t: (0, 0, 0)),
        ],
        out_specs=pl.BlockSpec((1, BB, L, D), lambda i, t: (t, i, 0, 0)),
        out_shape=jax.ShapeDtypeStruct((4, B, L, D), jnp.float32),
        scratch_shapes=[pltpu.VMEM((BB, LP, D), jnp.float32)],
        compiler_params=pltpu.CompilerParams(
            dimension_semantics=("arbitrary", "arbitrary"),
        ),
    )(rows3, pos, gamma, beta)


def kernel(x, emb_table, pos_embed, ln_gamma, ln_beta):
    fill = (jnp.arange(B * (LP - L), dtype=jnp.int32) % VOCAB).reshape(B, LP - L)
    xp = jnp.concatenate([x.astype(jnp.int32), fill], axis=1).reshape(-1)
    rows = _sc_gather(xp, emb_table)
    rows3 = rows.reshape(B, LP, D)
    pos = jnp.pad(pos_embed[:, :L, :], ((0, 0), (0, LP - L), (0, 0)))
    gam = ln_gamma.reshape(1, 1, D)
    bet = ln_beta.reshape(1, 1, D)
    return _tc_lif(rows3, pos, gam, bet)


# speedup vs baseline: 1.4440x; 1.4440x over previous
import functools

import jax
import jax.numpy as jnp
from jax import lax
from jax.experimental import pallas as pl
from jax.experimental.pallas import tpu as pltpu
from jax.experimental.pallas import tpu_sc as plsc

B, L, D = 1024, 50, 128
LP = 56
NP = B * LP
VOCAB = 100000

NC, NS = 2, 16
NW = NC * NS

K = 2
BK = B // K
NPK = NP // K
TOK_PER_W = NPK // NW
CHUNK = 64
NCHUNK = TOK_PER_W // CHUNK

EPS = 1e-5
C1, C2, C3, C4 = 2.0, 4.0 / 3.0, 8.0 / 7.0, 16.0 / 15.0


def _gather_body(x_hbm, table_hbm, out_hbm, idx_v, rows_v, sem0, sem1):
    wid = lax.axis_index("s") * NC + lax.axis_index("c")
    base = wid * TOK_PER_W

    def start(j, slot):
        off = base + j * CHUNK
        pltpu.sync_copy(x_hbm.at[pl.ds(off, CHUNK)], idx_v.at[slot])
        sem = sem0 if slot == 0 else sem1
        return pltpu.async_copy(table_hbm.at[idx_v.at[slot]], rows_v.at[slot], sem)

    dma = start(0, 0)
    for j in range(NCHUNK):
        slot = j % 2
        if j + 1 < NCHUNK:
            nxt = start(j + 1, (j + 1) % 2)
        dma.wait()
        off = base + j * CHUNK
        pltpu.sync_copy(rows_v.at[slot], out_hbm.at[pl.ds(off, CHUNK)])
        if j + 1 < NCHUNK:
            dma = nxt


def _sc_gather(x_flat, table):
    mesh = plsc.VectorSubcoreMesh(core_axis_name="c", subcore_axis_name="s")
    fn = pl.kernel(
        _gather_body,
        mesh=mesh,
        out_type=jax.ShapeDtypeStruct((NPK, D), jnp.float32),
        scratch_types=[
            pltpu.VMEM((2, CHUNK), jnp.int32),
            pltpu.VMEM((2, CHUNK, D), jnp.float32),
            pltpu.SemaphoreType.DMA,
            pltpu.SemaphoreType.DMA,
        ],
    )
    return fn(x_flat, table)


BB = 128


def _lif_body(rows_ref, pos_ref, gam_ref, bet_ref, prev_ref, out_ref):
    h = rows_ref[...] + pos_ref[...]
    mu = jnp.mean(h, axis=-1, keepdims=True)
    var = jnp.mean((h - mu) ** 2, axis=-1, keepdims=True)
    y = (h - mu) * lax.rsqrt(var + EPS) * gam_ref[...] + bet_ref[...]
    a = y >= C1
    b = y >= C2
    c = y >= C3
    d = y >= C4
    one = jnp.float32(1.0)
    zero = jnp.float32(0.0)
    out_ref[0] = jnp.where(a, one, zero)[:, :L, :]
    out_ref[1] = jnp.where(b, one, zero)[:, :L, :]
    out_ref[2] = jnp.where(a | (c & ~b), one, zero)[:, :L, :]
    out_ref[3] = jnp.where(b | (d & ~c), one, zero)[:, :L, :]


def _tc_lif_chunk(k, rows3, pos, gamma, beta, prev):
    boff = (k * BK) // BB
    grid = (BK // BB,)
    return pl.pallas_call(
        _lif_body,
        grid=grid,
        in_specs=[
            pl.BlockSpec((BB, LP, D), lambda i: (i, 0, 0)),
            pl.BlockSpec((1, LP, D), lambda i: (0, 0, 0)),
            pl.BlockSpec((1, 1, D), lambda i: (0, 0, 0)),
            pl.BlockSpec((1, 1, D), lambda i: (0, 0, 0)),
            pl.BlockSpec(memory_space=pltpu.MemorySpace.HBM),
        ],
        out_specs=pl.BlockSpec((4, BB, L, D), lambda i: (0, boff + i, 0, 0)),
        out_shape=jax.ShapeDtypeStruct((4, B, L, D), jnp.float32),
        input_output_aliases={4: 0},
        compiler_params=pltpu.CompilerParams(
            dimension_semantics=("arbitrary",),
        ),
    )(rows3, pos, gamma, beta, prev)


def kernel(x, emb_table, pos_embed, ln_gamma, ln_beta):
    fill = (jnp.arange(B * (LP - L), dtype=jnp.int32) % VOCAB).reshape(B, LP - L)
    xp = jnp.concatenate([x.astype(jnp.int32), fill], axis=1).reshape(K, NPK)
    pos = jnp.pad(pos_embed[:, :L, :], ((0, 0), (0, LP - L), (0, 0)))
    gam = ln_gamma.reshape(1, 1, D)
    bet = ln_beta.reshape(1, 1, D)

    rows = [_sc_gather(xp[k], emb_table).reshape(BK, LP, D) for k in range(K)]
    out = _tc_lif_first(rows[0], pos, gam, bet)
    for k in range(1, K):
        out = _tc_lif_chunk(k, rows[k], pos, gam, bet, out)
    return out


def _tc_lif_first(rows3, pos, gamma, beta):
    grid = (BK // BB,)
    return pl.pallas_call(
        lambda r, p, g, bt, o: _lif_body(r, p, g, bt, None, o),
        grid=grid,
        in_specs=[
            pl.BlockSpec((BB, LP, D), lambda i: (i, 0, 0)),
            pl.BlockSpec((1, LP, D), lambda i: (0, 0, 0)),
            pl.BlockSpec((1, 1, D), lambda i: (0, 0, 0)),
            pl.BlockSpec((1, 1, D), lambda i: (0, 0, 0)),
        ],
        out_specs=pl.BlockSpec((4, BB, L, D), lambda i: (0, i, 0, 0)),
        out_shape=jax.ShapeDtypeStruct((4, B, L, D), jnp.float32),
        compiler_params=pltpu.CompilerParams(
            dimension_semantics=("arbitrary",),
        ),
    )(rows3, pos, gamma, beta)
